# Initial kernel scaffold; baseline (speedup 1.0000x reference)
import jax
import jax.numpy as jnp
from jax.experimental import pallas as pl


def kernel(query, fts, Wq1, bq1, Wq2, bq2, Wk1, bk1, Wk2, bk2, Wv1, bv1, Wv2, bv2, Wo1, bo1, Wo2, bo2, Wf1, bf1, Wf2, bf2):
    raise NotImplementedError("write your pallas kernel here")


# speedup vs baseline: 2.2311x; 2.2311x over previous
import jax
import jax.numpy as jnp
from jax.experimental import pallas as pl

_B, _N, _M, _QD, _VD, _OUT = 4, 4096, 4096, 256, 256, 256
_SCALE = _OUT ** (-0.5)
_BN = 512
_BM = 512


def _lk(x):
    return jnp.where(x >= 0, x, 0.01 * x)


def _gl(x):
    return 0.5 * x * (1.0 + jax.lax.erf(x * (2.0 ** -0.5)))


def _kv_body(fts_ref, wk1t, bk1, wk2t, bk2, wv1t, bv1, wv2t, bv2, k_ref, v_ref):
    f = fts_ref[0]
    hk = _lk(jnp.dot(wk1t[...], f, preferred_element_type=jnp.float32) + bk1[...])
    k_ref[0] = _gl(jnp.dot(wk2t[...], hk, preferred_element_type=jnp.float32) + bk2[...])
    hv = _lk(jnp.dot(wv1t[...], f, preferred_element_type=jnp.float32) + bv1[...])
    v_ref[0] = _gl(jnp.dot(wv2t[...], hv, preferred_element_type=jnp.float32) + bv2[...])


def _attn_body(x_ref, k_ref, v_ref, wq1, bq1, wq2, bq2, wo1, bo1, wo2, bo2,
               wf1, bf1, wf2, bf2, out_ref):
    x = x_ref[0]
    h = _lk(jnp.dot(x, wq1[...], preferred_element_type=jnp.float32) + bq1[...])
    q = _gl(jnp.dot(h, wq2[...], preferred_element_type=jnp.float32) + bq2[...])
    s = jnp.dot(q, k_ref[0], preferred_element_type=jnp.float32) * _SCALE
    s = s - jnp.max(s, axis=1, keepdims=True)
    e = jnp.exp(s)
    p = e / jnp.sum(e, axis=1, keepdims=True)
    enh = jax.lax.dot_general(p, v_ref[0], (((1,), (1,)), ((), ())),
                              preferred_element_type=jnp.float32)
    h2 = _lk(jnp.dot(enh, wo1[...], preferred_element_type=jnp.float32) + bo1[...])
    ho = _gl(jnp.dot(h2, wo2[...], preferred_element_type=jnp.float32) + bo2[...])
    h3 = _lk(jnp.dot(ho, wf1[...], preferred_element_type=jnp.float32) + bf1[...])
    out_ref[0] = _lk(jnp.dot(h3, wf2[...], preferred_element_type=jnp.float32) + bf2[...])


def kernel(query, fts, Wq1, bq1, Wq2, bq2, Wk1, bk1, Wk2, bk2, Wv1, bv1,
           Wv2, bv2, Wo1, bo1, Wo2, bo2, Wf1, bf1, Wf2, bf2):
    col = lambda b: b.reshape(-1, 1)
    row = lambda b: b.reshape(1, -1)
    wspec = pl.BlockSpec((_QD, _OUT), lambda *_: (0, 0))
    cspec = pl.BlockSpec((_OUT, 1), lambda *_: (0, 0))
    rspec = pl.BlockSpec((1, _OUT), lambda *_: (0, 0))

    k_cm, v_cm = pl.pallas_call(
        _kv_body,
        grid=(_B, _M // _BM),
        in_specs=[
            pl.BlockSpec((1, _VD, _BM), lambda b, j: (b, 0, j)),
            wspec, cspec, wspec, cspec, wspec, cspec, wspec, cspec,
        ],
        out_specs=[
            pl.BlockSpec((1, _OUT, _BM), lambda b, j: (b, 0, j)),
            pl.BlockSpec((1, _OUT, _BM), lambda b, j: (b, 0, j)),
        ],
        out_shape=[
            jax.ShapeDtypeStruct((_B, _OUT, _M), jnp.float32),
            jax.ShapeDtypeStruct((_B, _OUT, _M), jnp.float32),
        ],
    )(fts, Wk1.T, col(bk1), Wk2.T, col(bk2), Wv1.T, col(bv1), Wv2.T, col(bv2))

    out = pl.pallas_call(
        _attn_body,
        grid=(_B, _N // _BN),
        in_specs=[
            pl.BlockSpec((1, _BN, _QD), lambda b, i: (b, i, 0)),
            pl.BlockSpec((1, _OUT, _M), lambda b, i: (b, 0, 0)),
            pl.BlockSpec((1, _OUT, _M), lambda b, i: (b, 0, 0)),
            wspec, rspec, wspec, rspec, wspec, rspec, wspec, rspec,
            wspec, rspec, wspec, rspec,
        ],
        out_specs=pl.BlockSpec((1, _BN, _OUT), lambda b, i: (b, i, 0)),
        out_shape=jax.ShapeDtypeStruct((_B, _N, _OUT), jnp.float32),
    )(query, k_cm, v_cm, Wq1, row(bq1), Wq2, row(bq2), Wo1, row(bo1),
      Wo2, row(bo2), Wf1, row(bf1), Wf2, row(bf2))
    return out
